# Initial kernel scaffold; baseline (speedup 1.0000x reference)
import jax
import jax.numpy as jnp
from jax.experimental import pallas as pl


def kernel(quant_fn, x, emb_idx, W1, b1, W2, b2):
    raise NotImplementedError("write your pallas kernel here")


# speedup vs baseline: 10.7599x; 10.7599x over previous
import functools

import jax
import jax.numpy as jnp
from jax.experimental import pallas as pl

B = 2048
X_SIZE = 128
H_SIZE = 128
OUT_SIZE = 128
NUM_EMB = 16


def _expert_step(x_ref, idx_ref, w1_ref, b1_ref, w2_ref, b2_ref, out_ref):
    e = pl.program_id(0)

    @pl.when(e == 0)
    def _init():
        out_ref[...] = jnp.zeros_like(out_ref)

    x = x_ref[...]
    w1 = w1_ref[0]
    h = jax.lax.dot_general(
        x, w1, (((1,), (1,)), ((), ())),
        preferred_element_type=jnp.float32)
    h = jnp.maximum(h + b1_ref[0], 0.0)
    w2 = w2_ref[0]
    y = jax.lax.dot_general(
        h, w2, (((1,), (1,)), ((), ())),
        preferred_element_type=jnp.float32)
    y = y + b2_ref[0]
    mask = (idx_ref[...] == e).astype(jnp.float32)
    out_ref[...] += mask * y


@functools.partial(jax.jit, static_argnames=())
def _run(x, emb_idx2d, W1, b1, W2, b2):
    return pl.pallas_call(
        _expert_step,
        grid=(NUM_EMB,),
        in_specs=[
            pl.BlockSpec((B, X_SIZE), lambda e: (0, 0)),
            pl.BlockSpec((B, 1), lambda e: (0, 0)),
            pl.BlockSpec((1, H_SIZE, X_SIZE), lambda e: (e, 0, 0)),
            pl.BlockSpec((1, 1, H_SIZE), lambda e: (e, 0, 0)),
            pl.BlockSpec((1, OUT_SIZE, H_SIZE), lambda e: (e, 0, 0)),
            pl.BlockSpec((1, 1, OUT_SIZE), lambda e: (e, 0, 0)),
        ],
        out_specs=pl.BlockSpec((B, OUT_SIZE), lambda e: (0, 0)),
        out_shape=jax.ShapeDtypeStruct((B, OUT_SIZE), jnp.float32),
    )(x, emb_idx2d, W1, b1, W2, b2)


def kernel(quant_fn, x, emb_idx, W1, b1, W2, b2):
    del quant_fn
    emb_idx2d = emb_idx.reshape(B, 1)
    b1r = b1.reshape(NUM_EMB, 1, H_SIZE)
    b2r = b2.reshape(NUM_EMB, 1, OUT_SIZE)
    return _run(x, emb_idx2d, W1, b1r, W2, b2r)
